# baseline (speedup 1.0000x reference)
import functools

import jax
import jax.numpy as jnp
from jax import lax
from jax.experimental import pallas as pl
from jax.experimental.pallas import tpu as pltpu
from jax.experimental.pallas import tpu_sc as plsc

N = 10000
E = 320000
D = 128
PW = 16

NC = 2
NS = 16
NW = NC * NS
SEG = 5
ES = E // SEG
EW = ES // NW
C = 80
NCHUNK = EW // C
RPS = N // NS
DN = 25

EB = 6400
NB = 2000


def _silu(v):
    return v * jax.nn.sigmoid(v)


def _dot(a, b):
    return jnp.dot(a, b, preferred_element_type=jnp.float32)


def _pq_body(x_ref, a_ref, b_ref, p_ref, q_ref):
    xb = x_ref[:]
    p_ref[:] = _dot(xb, a_ref[:])
    q_ref[:] = _dot(xb, b_ref[:])


def _tc_precompute(x, A, B):
    return pl.pallas_call(
        _pq_body,
        grid=(N // NB,),
        in_specs=[
            pl.BlockSpec((NB, D), lambda i: (i, 0)),
            pl.BlockSpec((D, D), lambda i: (0, 0)),
            pl.BlockSpec((D, D), lambda i: (0, 0)),
        ],
        out_specs=[
            pl.BlockSpec((NB, D), lambda i: (i, 0)),
            pl.BlockSpec((NB, D), lambda i: (i, 0)),
        ],
        out_shape=[
            jax.ShapeDtypeStruct((N, D), jnp.float32),
            jax.ShapeDtypeStruct((N, D), jnp.float32),
        ],
    )(x, A, B)


def _gather_body(row_hbm, col_hbm, p_hbm, q_hbm, pos_hbm,
                 os0, odist, orx, ory, orz,
                 rowv0, colv0, bufp0, bufq0, relb0, rxb0, ryb0, rzb0,
                 rowv1, colv1, bufp1, bufq1, relb1, rxb1, ryb1, rzb1,
                 posv,
                 sgp0, sgq0, sws0, swr0, srx0, sry0, srz0,
                 sgp1, sgq1, sws1, swr1, srx1, sry1, srz1):
    cid = lax.axis_index("c")
    sid = lax.axis_index("s")
    wid = sid * NC + cid
    base0 = wid * EW
    rowv = (rowv0, rowv1)
    colv = (colv0, colv1)
    bufp = (bufp0, bufp1)
    bufq = (bufq0, bufq1)
    relb = (relb0, relb1)
    sgp = (sgp0, sgp1)
    sgq = (sgq0, sgq1)
    sws = (sws0, sws1)
    swr = (swr0, swr1)
    relo = ((rxb0, ryb0, rzb0), (rxb1, ryb1, rzb1))
    srel = ((srx0, sry0, srz0), (srx1, sry1, srz1))
    orel3 = (orx, ory, orz)
    z16 = jnp.zeros((16,), jnp.float32)

    pltpu.sync_copy(pos_hbm, posv)

    def prefetch(j, b):
        base = base0 + j * C
        pltpu.sync_copy(row_hbm.at[pl.ds(base, C)], rowv[b])
        pltpu.sync_copy(col_hbm.at[pl.ds(base, C)], colv[b])
        pltpu.async_copy(p_hbm.at[rowv[b]], bufp[b], sgp[b])
        pltpu.async_copy(q_hbm.at[colv[b]], bufq[b], sgq[b])

    def wait_gathers(b):
        pltpu.make_async_copy(p_hbm.at[rowv[b]], bufp[b], sgp[b]).wait()
        pltpu.make_async_copy(q_hbm.at[colv[b]], bufq[b], sgq[b]).wait()

    def wait_writes(j, b):
        base = base0 + j * C
        pltpu.make_async_copy(bufp[b], os0.at[pl.ds(base, C)], sws[b]).wait()
        pltpu.make_async_copy(relb[b], odist.at[pl.ds(base, C)], swr[b]).wait()
        for kk in range(3):
            pltpu.make_async_copy(relo[b][kk], orel3[kk].at[pl.ds(base, C)],
                                  srel[b][kk]).wait()

    def compute_and_write(j, b):
        def srow(r, _):
            for d in range(D // 16):
                sl = pl.ds(d * 16, 16)
                bufp[b][r, sl] = bufp[b][r, sl] + bufq[b][r, sl]
            return 0
        lax.fori_loop(0, C, srow, 0)
        def grp(g, _):
            sl = pl.ds(g * 16, 16)
            ridx = rowv[b][sl]
            cidx = colv[b][sl]
            acc = jnp.zeros((16,), jnp.float32)
            for kk in range(3):
                kv = jnp.full((16,), kk, jnp.int32)
                a = plsc.load_gather(posv, [ridx, kv])
                c = plsc.load_gather(posv, [cidx, kv])
                r = a - c
                relo[b][kk][sl] = r
                acc = acc + r * r
            relb[b][sl] = acc
            return 0
        lax.fori_loop(0, C // 16, grp, 0)
        base = base0 + j * C
        pltpu.async_copy(bufp[b], os0.at[pl.ds(base, C)], sws[b])
        pltpu.async_copy(relb[b], odist.at[pl.ds(base, C)], swr[b])
        for kk in range(3):
            pltpu.async_copy(relo[b][kk], orel3[kk].at[pl.ds(base, C)],
                             srel[b][kk])

    prefetch(0, 0)

    def outer(k, _):
        j0 = 2 * k
        @pl.when(k > 0)
        def _():
            wait_writes(j0 - 1, 1)
        prefetch(j0 + 1, 1)
        wait_gathers(0)
        compute_and_write(j0, 0)
        wait_writes(j0, 0)
        prefetch(j0 + 2, 0)
        wait_gathers(1)
        compute_and_write(j0 + 1, 1)
        return 0

    lax.fori_loop(0, NCHUNK // 2, outer, 0)
    wait_gathers(0)
    compute_and_write(NCHUNK - 1, 0)
    wait_writes(NCHUNK - 2, 1)
    wait_writes(NCHUNK - 1, 0)


def _sc_gather(row, col, P, Q, pospad):
    f32 = jnp.float32
    i32 = jnp.int32
    slot = [
        pltpu.VMEM((C,), i32),
        pltpu.VMEM((C,), i32),
        pltpu.VMEM((C, D), f32),
        pltpu.VMEM((C, D), f32),
        pltpu.VMEM((C,), f32),
        pltpu.VMEM((C,), f32),
        pltpu.VMEM((C,), f32),
        pltpu.VMEM((C,), f32),
    ]
    sems = [pltpu.SemaphoreType.DMA] * 7
    k = functools.partial(
        pl.kernel,
        out_type=(
            jax.ShapeDtypeStruct((ES, D), f32),
            jax.ShapeDtypeStruct((ES,), f32),
            jax.ShapeDtypeStruct((ES,), f32),
            jax.ShapeDtypeStruct((ES,), f32),
            jax.ShapeDtypeStruct((ES,), f32),
        ),
        compiler_params=pltpu.CompilerParams(use_tc_tiling_on_sc=False,
                                             needs_layout_passes=False),
        mesh=plsc.VectorSubcoreMesh(core_axis_name="c", subcore_axis_name="s"),
        scratch_types=slot + slot + [pltpu.VMEM((N, 4), f32)] + sems + sems,
    )(_gather_body)
    return k(row, col, P, Q, pospad)


RB = EB // D


def _edge_body(s0_ref, dist_ref, rx_ref, ry_ref, rz_ref,
               wlast_ref, be1_ref, we2_ref, be2_ref,
               wc1_ref, bc1_ref, wc2_ref,
               em_ref, ax_ref, ay_ref, az_ref):
    x1 = s0_ref[:]
    dp = dist_ref[0]
    dpt = lax.transpose(dp, (1, 0))
    dist = jnp.concatenate(
        [dpt[:, r:r + 1] for r in range(RB)], axis=0)
    h = _silu(x1 + dist * wlast_ref[:] + be1_ref[:])
    em = _silu(_dot(h, we2_ref[:]) + be2_ref[:])
    t = _silu(_dot(em, wc1_ref[:]) + bc1_ref[:])
    cw = _dot(t, wc2_ref[:])
    cwm = jnp.concatenate(
        [cw[D * r:D * r + D, :] for r in range(RB)], axis=1)
    cwp = lax.transpose(cwm, (1, 0))
    em_ref[:] = em
    ax_ref[0] = cwp * rx_ref[0]
    ay_ref[0] = cwp * ry_ref[0]
    az_ref[0] = cwp * rz_ref[0]


def _tc_edge(s0, distp, rxp, ryp, rzp, wlast, be1, We2, be2, Wc1, bc1, wc2c):
    w0 = lambda i: (0, 0)
    pk = pl.BlockSpec((1, RB, D), lambda i: (i, 0, 0))
    return pl.pallas_call(
        _edge_body,
        grid=(ES // EB,),
        in_specs=[
            pl.BlockSpec((EB, D), lambda i: (i, 0)),
            pk, pk, pk, pk,
            pl.BlockSpec((1, D), w0),
            pl.BlockSpec((1, D), w0),
            pl.BlockSpec((D, D), w0),
            pl.BlockSpec((1, D), w0),
            pl.BlockSpec((D, D), w0),
            pl.BlockSpec((1, D), w0),
            pl.BlockSpec((D, 1), w0),
        ],
        out_specs=[
            pl.BlockSpec((EB, D), lambda i: (i, 0)),
            pk, pk, pk,
        ],
        out_shape=[
            jax.ShapeDtypeStruct((ES, D), jnp.float32),
            jax.ShapeDtypeStruct((ES // EB, RB, D), jnp.float32),
            jax.ShapeDtypeStruct((ES // EB, RB, D), jnp.float32),
            jax.ShapeDtypeStruct((ES // EB, RB, D), jnp.float32),
        ],
    )(s0, distp, rxp, ryp, rzp, wlast, be1, We2, be2, Wc1, bc1, wc2c)


def _scatter_body(row_hbm, em_hbm, rx_hbm, ry_hbm, rz_hbm,
                  agg_out, aux_out,
                  idxv0, emv0, rxv0, ryv0, rzv0,
                  idxv1, emv1, rxv1, ryv1, rzv1,
                  auxu, db, dba, agg_s, aux_s,
                  se0, sx0, sy0, sz0,
                  se1, sx1, sy1, sz1):
    cid = lax.axis_index("c")
    sid = lax.axis_index("s")
    wid = sid * NC + cid
    base0 = wid * EW
    z16 = jnp.zeros((16,), jnp.float32)
    iota16 = lax.iota(jnp.int32, 16)
    e3row = (iota16 == 3).astype(jnp.float32)

    def initaux(i, _):
        auxu[i, :] = e3row
        return 0

    lax.fori_loop(0, C, initaux, 0)

    def zrow(i, _):
        def zcol(j, _):
            db[i, pl.ds(j * 16, 16)] = z16
            return 0
        lax.fori_loop(0, D // 16, zcol, 0)
        dba[i, :] = z16
        return 0

    lax.fori_loop(0, DN, zrow, 0)

    r0 = sid * RPS

    def zdump(j, _):
        pltpu.sync_copy(db, agg_s.at[pl.ds(r0 + j * DN, DN)])
        pltpu.sync_copy(dba, aux_s.at[pl.ds(r0 + j * DN, DN)])
        return 0

    lax.fori_loop(0, RPS // DN, zdump, 0)
    plsc.subcore_barrier()

    idxv = (idxv0, idxv1)
    emv = (emv0, emv1)
    relv = ((rxv0, ryv0, rzv0), (rxv1, ryv1, rzv1))
    rel_hbm = (rx_hbm, ry_hbm, rz_hbm)
    se = (se0, se1)
    srel = ((sx0, sy0, sz0), (sx1, sy1, sz1))

    def prefetch(j, b):
        base = base0 + j * C
        pltpu.sync_copy(row_hbm.at[pl.ds(base, C)], idxv[b])
        pltpu.async_copy(em_hbm.at[pl.ds(base, C)], emv[b], se[b])
        for kk in range(3):
            pltpu.async_copy(rel_hbm[kk].at[pl.ds(base, C)], relv[b][kk],
                             srel[b][kk])

    def scatter(j, b):
        base = base0 + j * C
        pltpu.make_async_copy(em_hbm.at[pl.ds(base, C)], emv[b], se[b]).wait()
        for kk in range(3):
            pltpu.make_async_copy(rel_hbm[kk].at[pl.ds(base, C)], relv[b][kk],
                                  srel[b][kk]).wait()
        def grp(g, _, b=b):
            sl = pl.ds(g * 16, 16)
            eidx = g * 16 + iota16
            for kk in range(3):
                kv = jnp.full((16,), kk, jnp.int32)
                plsc.store_scatter(auxu, [eidx, kv], relv[b][kk][sl])
            return 0
        lax.fori_loop(0, C // 16, grp, 0)
        pltpu.sync_copy(emv[b], agg_s.at[idxv[b]], add=True)
        pltpu.sync_copy(auxu, aux_s.at[idxv[b]], add=True)

    prefetch(0, 0)

    def chunk(k, _):
        j0 = 2 * k
        prefetch(j0 + 1, 1)
        scatter(j0, 0)
        prefetch(j0 + 2, 0)
        scatter(j0 + 1, 1)
        return 0

    lax.fori_loop(0, NCHUNK // 2, chunk, 0)
    scatter(NCHUNK - 1, 0)
    plsc.subcore_barrier()

    def dump(j, _):
        r = r0 + j * DN
        pltpu.sync_copy(agg_s.at[pl.ds(r, DN)], db)
        pltpu.sync_copy(db, agg_out.at[cid].at[pl.ds(r, DN)])
        pltpu.sync_copy(aux_s.at[pl.ds(r, DN)], dba)
        pltpu.sync_copy(dba, aux_out.at[cid].at[pl.ds(r, DN)])
        return 0

    lax.fori_loop(0, RPS // DN, dump, 0)


def _sc_scatter(row, em, ax, ay, az):
    f32 = jnp.float32
    k = functools.partial(
        pl.kernel,
        out_type=(
            jax.ShapeDtypeStruct((NC, N, D), f32),
            jax.ShapeDtypeStruct((NC, N, PW), f32),
        ),
        compiler_params=pltpu.CompilerParams(use_tc_tiling_on_sc=False,
                                             needs_layout_passes=False,
                                             internal_scratch_in_bytes=1 << 18),
        mesh=plsc.VectorSubcoreMesh(core_axis_name="c", subcore_axis_name="s"),
        scratch_types=(
            [pltpu.VMEM((C,), jnp.int32),
             pltpu.VMEM((C, D), f32),
             pltpu.VMEM((C,), f32),
             pltpu.VMEM((C,), f32),
             pltpu.VMEM((C,), f32)] * 2
            + [pltpu.VMEM((C, PW), f32),
               pltpu.VMEM((DN, D), f32),
               pltpu.VMEM((DN, PW), f32),
               pltpu.VMEM_SHARED((N, D), f32),
               pltpu.VMEM_SHARED((N, PW), f32)]
            + [pltpu.SemaphoreType.DMA] * 8
        ),
    )(_scatter_body)
    return k(row, em, ax, ay, az)


def _node_body(x_ref, a0, x0, a1, x1r, a2, x2, a3, x3, a4, x4, pos_ref,
               wn1a_ref, wn1b_ref, bn1_ref, wn2_ref, bn2_ref,
               g_ref, b_ref,
               xn_ref, pn_ref):
    xb = x_ref[:]
    aggs = (a0, a1, a2, a3, a4)
    auxs = (x0, x1r, x2, x3, x4)
    agg = aggs[0][0] + aggs[0][1]
    aux = auxs[0][0] + auxs[0][1]
    for t in range(1, SEG):
        agg = agg + aggs[t][0] + aggs[t][1]
        aux = aux + auxs[t][0] + auxs[t][1]
    cnt = aux[:, 3:4]
    inv = 1.0 / (cnt + 1e-6)
    pn_ref[:] = pos_ref[:] + aux * inv
    u = _silu(_dot(xb, wn1a_ref[:]) + _dot(agg, wn1b_ref[:]) + bn1_ref[:])
    v = xb + _dot(u, wn2_ref[:]) + bn2_ref[:]
    mu = jnp.mean(v, axis=1, keepdims=True)
    vc = v - mu
    var = jnp.mean(vc * vc, axis=1, keepdims=True)
    xn_ref[:] = vc * lax.rsqrt(var + 1e-5) * g_ref[:] + b_ref[:]


def _tc_node(x, parts, pospad,
             Wn1a, Wn1b, bn1, Wn2, bn2, gamma, beta):
    w0 = lambda i: (0, 0)
    pspecs = []
    flat = []
    for t in range(SEG):
        pspecs.append(pl.BlockSpec((NC, NB, D), lambda i: (0, i, 0)))
        pspecs.append(pl.BlockSpec((NC, NB, PW), lambda i: (0, i, 0)))
        flat.extend([parts[t][0], parts[t][1]])
    return pl.pallas_call(
        _node_body,
        grid=(N // NB,),
        in_specs=[
            pl.BlockSpec((NB, D), lambda i: (i, 0)),
            *pspecs,
            pl.BlockSpec((NB, PW), lambda i: (i, 0)),
            pl.BlockSpec((D, D), w0),
            pl.BlockSpec((D, D), w0),
            pl.BlockSpec((1, D), w0),
            pl.BlockSpec((D, D), w0),
            pl.BlockSpec((1, D), w0),
            pl.BlockSpec((1, D), w0),
            pl.BlockSpec((1, D), w0),
        ],
        out_specs=[
            pl.BlockSpec((NB, D), lambda i: (i, 0)),
            pl.BlockSpec((NB, PW), lambda i: (i, 0)),
        ],
        out_shape=[
            jax.ShapeDtypeStruct((N, D), jnp.float32),
            jax.ShapeDtypeStruct((N, PW), jnp.float32),
        ],
    )(x, *flat, pospad,
      Wn1a, Wn1b, bn1, Wn2, bn2, gamma, beta)


def kernel(x, pos, edge_index, We1, be1, We2, be2, Wn1, bn1, Wn2, bn2,
           Wc1, bc1, Wc2, gamma, beta):
    row = edge_index[0]
    col = edge_index[1]
    pospad4 = jnp.pad(pos, ((0, 0), (0, 1)))
    pospad = jnp.pad(pos, ((0, 0), (0, PW - 3)))
    A = We1[:D]
    B = We1[D:2 * D]
    wlast = We1[2 * D:2 * D + 1]
    P, Q = _tc_precompute(x, A, B)
    pshape = (ES // EB, RB, D)
    parts = []
    for sgi in range(SEG):
        row_s = lax.slice(row, (sgi * ES,), ((sgi + 1) * ES,))
        col_s = lax.slice(col, (sgi * ES,), ((sgi + 1) * ES,))
        s0, dist, rx, ry, rz = _sc_gather(row_s, col_s, P, Q, pospad4)
        em, axp, ayp, azp = _tc_edge(s0, dist.reshape(pshape),
                                     rx.reshape(pshape),
                                     ry.reshape(pshape),
                                     rz.reshape(pshape),
                                     wlast, be1.reshape(1, D), We2,
                                     be2.reshape(1, D), Wc1,
                                     bc1.reshape(1, D), Wc2)
        parts.append(_sc_scatter(row_s, em, axp.reshape(ES),
                                 ayp.reshape(ES), azp.reshape(ES)))
    xn, pn = _tc_node(x, parts, pospad,
                      Wn1[:D], Wn1[D:], bn1.reshape(1, D), Wn2,
                      bn2.reshape(1, D), gamma.reshape(1, D),
                      beta.reshape(1, D))
    return xn, pn[:, :3]


# speedup vs baseline: 1.1117x; 1.1117x over previous
import functools

import jax
import jax.numpy as jnp
from jax import lax
from jax.experimental import pallas as pl
from jax.experimental.pallas import tpu as pltpu
from jax.experimental.pallas import tpu_sc as plsc

N = 10000
E = 320000
D = 128
PW = 16

NC = 2
NS = 16
NW = NC * NS
SEG = 5
ES = E // SEG
EW = ES // NW
C = 80
NCHUNK = EW // C
RPS = N // NS
DN = 25

EB = 6400
NB = 2000


def _silu(v):
    return v * jax.nn.sigmoid(v)


def _dot(a, b):
    return jnp.dot(a, b, preferred_element_type=jnp.float32)


def _pq_body(x_ref, a_ref, b_ref, p_ref, q_ref):
    xb = x_ref[:]
    p_ref[:] = _dot(xb, a_ref[:])
    q_ref[:] = _dot(xb, b_ref[:])


def _tc_precompute(x, A, B):
    return pl.pallas_call(
        _pq_body,
        grid=(N // NB,),
        in_specs=[
            pl.BlockSpec((NB, D), lambda i: (i, 0)),
            pl.BlockSpec((D, D), lambda i: (0, 0)),
            pl.BlockSpec((D, D), lambda i: (0, 0)),
        ],
        out_specs=[
            pl.BlockSpec((NB, D), lambda i: (i, 0)),
            pl.BlockSpec((NB, D), lambda i: (i, 0)),
        ],
        out_shape=[
            jax.ShapeDtypeStruct((N, D), jnp.float32),
            jax.ShapeDtypeStruct((N, D), jnp.float32),
        ],
    )(x, A, B)


def _gather_body(row_hbm, col_hbm, p_hbm, q_hbm, pos_hbm,
                 os0, odist, orx, ory, orz,
                 rowv0, colv0, bufp0, bufq0, relb0, rxb0, ryb0, rzb0,
                 rowv1, colv1, bufp1, bufq1, relb1, rxb1, ryb1, rzb1,
                 posv,
                 sgp0, sgq0, sws0, swr0, srx0, sry0, srz0,
                 sgp1, sgq1, sws1, swr1, srx1, sry1, srz1):
    cid = lax.axis_index("c")
    sid = lax.axis_index("s")
    wid = sid * NC + cid
    base0 = wid * EW
    rowv = (rowv0, rowv1)
    colv = (colv0, colv1)
    bufp = (bufp0, bufp1)
    bufq = (bufq0, bufq1)
    relb = (relb0, relb1)
    sgp = (sgp0, sgp1)
    sgq = (sgq0, sgq1)
    sws = (sws0, sws1)
    swr = (swr0, swr1)
    relo = ((rxb0, ryb0, rzb0), (rxb1, ryb1, rzb1))
    srel = ((srx0, sry0, srz0), (srx1, sry1, srz1))
    orel3 = (orx, ory, orz)
    z16 = jnp.zeros((16,), jnp.float32)

    pltpu.sync_copy(pos_hbm, posv)

    def prefetch(j, b):
        base = base0 + j * C
        pltpu.sync_copy(row_hbm.at[pl.ds(base, C)], rowv[b])
        pltpu.sync_copy(col_hbm.at[pl.ds(base, C)], colv[b])
        pltpu.async_copy(p_hbm.at[rowv[b]], bufp[b], sgp[b])
        pltpu.async_copy(q_hbm.at[colv[b]], bufq[b], sgq[b])

    def wait_gathers(b):
        pltpu.make_async_copy(p_hbm.at[rowv[b]], bufp[b], sgp[b]).wait()
        pltpu.make_async_copy(q_hbm.at[colv[b]], bufq[b], sgq[b]).wait()

    def wait_writes(j, b):
        base = base0 + j * C
        pltpu.make_async_copy(bufp[b], os0.at[pl.ds(base, C)], sws[b]).wait()
        pltpu.make_async_copy(relb[b], odist.at[pl.ds(base, C)], swr[b]).wait()
        for kk in range(3):
            pltpu.make_async_copy(relo[b][kk], orel3[kk].at[pl.ds(base, C)],
                                  srel[b][kk]).wait()

    def compute_and_write(j, b):
        def srow(r, _):
            for d in range(D // 16):
                sl = pl.ds(d * 16, 16)
                bufp[b][r, sl] = bufp[b][r, sl] + bufq[b][r, sl]
            return 0
        lax.fori_loop(0, C, srow, 0)
        def grp(g, _):
            sl = pl.ds(g * 16, 16)
            ridx = rowv[b][sl]
            cidx = colv[b][sl]
            acc = jnp.zeros((16,), jnp.float32)
            for kk in range(3):
                kv = jnp.full((16,), kk, jnp.int32)
                a = plsc.load_gather(posv, [ridx, kv])
                c = plsc.load_gather(posv, [cidx, kv])
                r = a - c
                relo[b][kk][sl] = r
                acc = acc + r * r
            relb[b][sl] = acc
            return 0
        lax.fori_loop(0, C // 16, grp, 0)
        base = base0 + j * C
        pltpu.async_copy(bufp[b], os0.at[pl.ds(base, C)], sws[b])
        pltpu.async_copy(relb[b], odist.at[pl.ds(base, C)], swr[b])
        for kk in range(3):
            pltpu.async_copy(relo[b][kk], orel3[kk].at[pl.ds(base, C)],
                             srel[b][kk])

    prefetch(0, 0)

    def outer(k, _):
        j0 = 2 * k
        @pl.when(k > 0)
        def _():
            wait_writes(j0 - 1, 1)
        prefetch(j0 + 1, 1)
        wait_gathers(0)
        compute_and_write(j0, 0)
        wait_writes(j0, 0)
        prefetch(j0 + 2, 0)
        wait_gathers(1)
        compute_and_write(j0 + 1, 1)
        return 0

    lax.fori_loop(0, NCHUNK // 2, outer, 0)
    wait_gathers(0)
    compute_and_write(NCHUNK - 1, 0)
    wait_writes(NCHUNK - 2, 1)
    wait_writes(NCHUNK - 1, 0)


def _sc_gather(row, col, P, Q, pospad):
    f32 = jnp.float32
    i32 = jnp.int32
    slot = [
        pltpu.VMEM((C,), i32),
        pltpu.VMEM((C,), i32),
        pltpu.VMEM((C, D), f32),
        pltpu.VMEM((C, D), f32),
        pltpu.VMEM((C,), f32),
        pltpu.VMEM((C,), f32),
        pltpu.VMEM((C,), f32),
        pltpu.VMEM((C,), f32),
    ]
    sems = [pltpu.SemaphoreType.DMA] * 7
    k = functools.partial(
        pl.kernel,
        out_type=(
            jax.ShapeDtypeStruct((ES, D), f32),
            jax.ShapeDtypeStruct((ES,), f32),
            jax.ShapeDtypeStruct((ES,), f32),
            jax.ShapeDtypeStruct((ES,), f32),
            jax.ShapeDtypeStruct((ES,), f32),
        ),
        compiler_params=pltpu.CompilerParams(use_tc_tiling_on_sc=False,
                                             needs_layout_passes=False),
        mesh=plsc.VectorSubcoreMesh(core_axis_name="c", subcore_axis_name="s"),
        scratch_types=slot + slot + [pltpu.VMEM((N, 4), f32)] + sems + sems,
    )(_gather_body)
    return k(row, col, P, Q, pospad)


RB = EB // D


def _edge_body(s0_ref, dist_ref, rx_ref, ry_ref, rz_ref,
               wlast_ref, be1_ref, we2_ref, be2_ref,
               wc1_ref, bc1_ref, wc2_ref,
               em_ref, ax_ref, ay_ref, az_ref):
    x1 = s0_ref[:]
    dp = dist_ref[0]
    dpt = lax.transpose(dp, (1, 0))
    dist = jnp.concatenate(
        [dpt[:, r:r + 1] for r in range(RB)], axis=0)
    h = _silu(x1 + dist * wlast_ref[:] + be1_ref[:])
    em = _silu(_dot(h, we2_ref[:]) + be2_ref[:])
    t = _silu(_dot(em, wc1_ref[:]) + bc1_ref[:])
    cw = _dot(t, wc2_ref[:])
    cwm = jnp.concatenate(
        [cw[D * r:D * r + D, :] for r in range(RB)], axis=1)
    cwp = lax.transpose(cwm, (1, 0))
    em_ref[:] = em
    ax_ref[0] = cwp * rx_ref[0]
    ay_ref[0] = cwp * ry_ref[0]
    az_ref[0] = cwp * rz_ref[0]


def _tc_edge(s0, distp, rxp, ryp, rzp, wlast, be1, We2, be2, Wc1, bc1, wc2c):
    w0 = lambda i: (0, 0)
    pk = pl.BlockSpec((1, RB, D), lambda i: (i, 0, 0))
    return pl.pallas_call(
        _edge_body,
        grid=(ES // EB,),
        in_specs=[
            pl.BlockSpec((EB, D), lambda i: (i, 0)),
            pk, pk, pk, pk,
            pl.BlockSpec((1, D), w0),
            pl.BlockSpec((1, D), w0),
            pl.BlockSpec((D, D), w0),
            pl.BlockSpec((1, D), w0),
            pl.BlockSpec((D, D), w0),
            pl.BlockSpec((1, D), w0),
            pl.BlockSpec((D, 1), w0),
        ],
        out_specs=[
            pl.BlockSpec((EB, D), lambda i: (i, 0)),
            pk, pk, pk,
        ],
        out_shape=[
            jax.ShapeDtypeStruct((ES, D), jnp.float32),
            jax.ShapeDtypeStruct((ES // EB, RB, D), jnp.float32),
            jax.ShapeDtypeStruct((ES // EB, RB, D), jnp.float32),
            jax.ShapeDtypeStruct((ES // EB, RB, D), jnp.float32),
        ],
    )(s0, distp, rxp, ryp, rzp, wlast, be1, We2, be2, Wc1, bc1, wc2c)


def _scatter_body(*refs):
    (row_hbm, em_hbm, rx_hbm, ry_hbm, rz_hbm) = [refs[5 * t:5 * t + 5]
                                                 for t in range(SEG)][0]
    seg_refs = [refs[5 * t:5 * t + 5] for t in range(SEG)]
    (agg_out, aux_out,
     idxv0, emv0, rxv0, ryv0, rzv0,
     idxv1, emv1, rxv1, ryv1, rzv1,
     auxu, db, dba, agg_s, aux_s,
     se0, sx0, sy0, sz0,
     se1, sx1, sy1, sz1) = refs[5 * SEG:]
    cid = lax.axis_index("c")
    sid = lax.axis_index("s")
    wid = sid * NC + cid
    base0 = wid * EW
    z16 = jnp.zeros((16,), jnp.float32)
    iota16 = lax.iota(jnp.int32, 16)
    e3row = (iota16 == 3).astype(jnp.float32)

    def initaux(i, _):
        auxu[i, :] = e3row
        return 0

    lax.fori_loop(0, C, initaux, 0)

    def zrow(i, _):
        def zcol(j, _):
            db[i, pl.ds(j * 16, 16)] = z16
            return 0
        lax.fori_loop(0, D // 16, zcol, 0)
        dba[i, :] = z16
        return 0

    lax.fori_loop(0, DN, zrow, 0)

    r0 = sid * RPS

    def zdump(j, _):
        pltpu.sync_copy(db, agg_s.at[pl.ds(r0 + j * DN, DN)])
        pltpu.sync_copy(dba, aux_s.at[pl.ds(r0 + j * DN, DN)])
        return 0

    lax.fori_loop(0, RPS // DN, zdump, 0)
    plsc.subcore_barrier()

    idxv = (idxv0, idxv1)
    emv = (emv0, emv1)
    relv = ((rxv0, ryv0, rzv0), (rxv1, ryv1, rzv1))
    se = (se0, se1)
    srel = ((sx0, sy0, sz0), (sx1, sy1, sz1))

    for t in range(SEG):
        (row_hbm, em_hbm, rx_hbm, ry_hbm, rz_hbm) = seg_refs[t]
        rel_hbm = (rx_hbm, ry_hbm, rz_hbm)

        def prefetch(j, b, row_hbm=row_hbm, em_hbm=em_hbm, rel_hbm=rel_hbm):
            base = base0 + j * C
            pltpu.sync_copy(row_hbm.at[pl.ds(base, C)], idxv[b])
            pltpu.async_copy(em_hbm.at[pl.ds(base, C)], emv[b], se[b])
            for kk in range(3):
                pltpu.async_copy(rel_hbm[kk].at[pl.ds(base, C)], relv[b][kk],
                                 srel[b][kk])

        def scatter(j, b, em_hbm=em_hbm, rel_hbm=rel_hbm):
            base = base0 + j * C
            pltpu.make_async_copy(em_hbm.at[pl.ds(base, C)], emv[b],
                                  se[b]).wait()
            for kk in range(3):
                pltpu.make_async_copy(rel_hbm[kk].at[pl.ds(base, C)],
                                      relv[b][kk], srel[b][kk]).wait()
            def grp(g, _, b=b):
                sl = pl.ds(g * 16, 16)
                eidx = g * 16 + iota16
                for kk in range(3):
                    kv = jnp.full((16,), kk, jnp.int32)
                    plsc.store_scatter(auxu, [eidx, kv], relv[b][kk][sl])
                return 0
            lax.fori_loop(0, C // 16, grp, 0)
            pltpu.sync_copy(emv[b], agg_s.at[idxv[b]], add=True)
            pltpu.sync_copy(auxu, aux_s.at[idxv[b]], add=True)

        prefetch(0, 0)

        def chunk(k, _, prefetch=prefetch, scatter=scatter):
            j0 = 2 * k
            prefetch(j0 + 1, 1)
            scatter(j0, 0)
            prefetch(j0 + 2, 0)
            scatter(j0 + 1, 1)
            return 0

        lax.fori_loop(0, NCHUNK // 2, chunk, 0)
        scatter(NCHUNK - 1, 0)
    plsc.subcore_barrier()

    def dump(j, _):
        r = r0 + j * DN
        pltpu.sync_copy(agg_s.at[pl.ds(r, DN)], db)
        pltpu.sync_copy(db, agg_out.at[cid].at[pl.ds(r, DN)])
        pltpu.sync_copy(aux_s.at[pl.ds(r, DN)], dba)
        pltpu.sync_copy(dba, aux_out.at[cid].at[pl.ds(r, DN)])
        return 0

    lax.fori_loop(0, RPS // DN, dump, 0)


def _sc_scatter(seg_args):
    f32 = jnp.float32
    k = functools.partial(
        pl.kernel,
        out_type=(
            jax.ShapeDtypeStruct((NC, N, D), f32),
            jax.ShapeDtypeStruct((NC, N, PW), f32),
        ),
        compiler_params=pltpu.CompilerParams(use_tc_tiling_on_sc=False,
                                             needs_layout_passes=False,
                                             internal_scratch_in_bytes=1 << 18),
        mesh=plsc.VectorSubcoreMesh(core_axis_name="c", subcore_axis_name="s"),
        scratch_types=(
            [pltpu.VMEM((C,), jnp.int32),
             pltpu.VMEM((C, D), f32),
             pltpu.VMEM((C,), f32),
             pltpu.VMEM((C,), f32),
             pltpu.VMEM((C,), f32)] * 2
            + [pltpu.VMEM((C, PW), f32),
               pltpu.VMEM((DN, D), f32),
               pltpu.VMEM((DN, PW), f32),
               pltpu.VMEM_SHARED((N, D), f32),
               pltpu.VMEM_SHARED((N, PW), f32)]
            + [pltpu.SemaphoreType.DMA] * 8
        ),
    )(_scatter_body)
    flat = [a for sa in seg_args for a in sa]
    return k(*flat)


def _node_body(x_ref, agg_ref, aux_ref, pos_ref,
               wn1a_ref, wn1b_ref, bn1_ref, wn2_ref, bn2_ref,
               g_ref, b_ref,
               xn_ref, pn_ref):
    xb = x_ref[:]
    agg = agg_ref[0] + agg_ref[1]
    aux = aux_ref[0] + aux_ref[1]
    cnt = aux[:, 3:4]
    inv = 1.0 / (cnt + 1e-6)
    pn_ref[:] = pos_ref[:] + aux * inv
    u = _silu(_dot(xb, wn1a_ref[:]) + _dot(agg, wn1b_ref[:]) + bn1_ref[:])
    v = xb + _dot(u, wn2_ref[:]) + bn2_ref[:]
    mu = jnp.mean(v, axis=1, keepdims=True)
    vc = v - mu
    var = jnp.mean(vc * vc, axis=1, keepdims=True)
    xn_ref[:] = vc * lax.rsqrt(var + 1e-5) * g_ref[:] + b_ref[:]


def _tc_node(x, aggP, auxP, pospad,
             Wn1a, Wn1b, bn1, Wn2, bn2, gamma, beta):
    w0 = lambda i: (0, 0)
    return pl.pallas_call(
        _node_body,
        grid=(N // NB,),
        in_specs=[
            pl.BlockSpec((NB, D), lambda i: (i, 0)),
            pl.BlockSpec((NC, NB, D), lambda i: (0, i, 0)),
            pl.BlockSpec((NC, NB, PW), lambda i: (0, i, 0)),
            pl.BlockSpec((NB, PW), lambda i: (i, 0)),
            pl.BlockSpec((D, D), w0),
            pl.BlockSpec((D, D), w0),
            pl.BlockSpec((1, D), w0),
            pl.BlockSpec((D, D), w0),
            pl.BlockSpec((1, D), w0),
            pl.BlockSpec((1, D), w0),
            pl.BlockSpec((1, D), w0),
        ],
        out_specs=[
            pl.BlockSpec((NB, D), lambda i: (i, 0)),
            pl.BlockSpec((NB, PW), lambda i: (i, 0)),
        ],
        out_shape=[
            jax.ShapeDtypeStruct((N, D), jnp.float32),
            jax.ShapeDtypeStruct((N, PW), jnp.float32),
        ],
    )(x, aggP, auxP, pospad,
      Wn1a, Wn1b, bn1, Wn2, bn2, gamma, beta)


def kernel(x, pos, edge_index, We1, be1, We2, be2, Wn1, bn1, Wn2, bn2,
           Wc1, bc1, Wc2, gamma, beta):
    row = edge_index[0]
    col = edge_index[1]
    pospad4 = jnp.pad(pos, ((0, 0), (0, 1)))
    pospad = jnp.pad(pos, ((0, 0), (0, PW - 3)))
    A = We1[:D]
    B = We1[D:2 * D]
    wlast = We1[2 * D:2 * D + 1]
    P, Q = _tc_precompute(x, A, B)
    pshape = (ES // EB, RB, D)
    seg_args = []
    for sgi in range(SEG):
        row_s = lax.slice(row, (sgi * ES,), ((sgi + 1) * ES,))
        col_s = lax.slice(col, (sgi * ES,), ((sgi + 1) * ES,))
        s0, dist, rx, ry, rz = _sc_gather(row_s, col_s, P, Q, pospad4)
        em, axp, ayp, azp = _tc_edge(s0, dist.reshape(pshape),
                                     rx.reshape(pshape),
                                     ry.reshape(pshape),
                                     rz.reshape(pshape),
                                     wlast, be1.reshape(1, D), We2,
                                     be2.reshape(1, D), Wc1,
                                     bc1.reshape(1, D), Wc2)
        seg_args.append((row_s, em, axp.reshape(ES), ayp.reshape(ES),
                         azp.reshape(ES)))
    aggP, auxP = _sc_scatter(seg_args)
    xn, pn = _tc_node(x, aggP, auxP, pospad,
                      Wn1[:D], Wn1[D:], bn1.reshape(1, D), Wn2,
                      bn2.reshape(1, D), gamma.reshape(1, D),
                      beta.reshape(1, D))
    return xn, pn[:, :3]


# speedup vs baseline: 1.1228x; 1.0100x over previous
import functools

import jax
import jax.numpy as jnp
from jax import lax
from jax.experimental import pallas as pl
from jax.experimental.pallas import tpu as pltpu
from jax.experimental.pallas import tpu_sc as plsc

N = 10000
E = 320000
D = 128
PW = 16

NC = 2
NS = 16
NW = NC * NS
SEG = 5
ES = E // SEG
EW = ES // NW
C = 80
NCHUNK = EW // C
RPS = N // NS
DN = 25

EB = 6400
NB = 2000


def _silu(v):
    return v * jax.nn.sigmoid(v)


def _dot(a, b):
    return jnp.dot(a, b, preferred_element_type=jnp.float32)


def _pq_body(x_ref, a_ref, b_ref, p_ref, q_ref):
    xb = x_ref[:]
    p_ref[:] = _dot(xb, a_ref[:])
    q_ref[:] = _dot(xb, b_ref[:])


def _tc_precompute(x, A, B):
    return pl.pallas_call(
        _pq_body,
        grid=(N // NB,),
        in_specs=[
            pl.BlockSpec((NB, D), lambda i: (i, 0)),
            pl.BlockSpec((D, D), lambda i: (0, 0)),
            pl.BlockSpec((D, D), lambda i: (0, 0)),
        ],
        out_specs=[
            pl.BlockSpec((NB, D), lambda i: (i, 0)),
            pl.BlockSpec((NB, D), lambda i: (i, 0)),
        ],
        out_shape=[
            jax.ShapeDtypeStruct((N, D), jnp.float32),
            jax.ShapeDtypeStruct((N, D), jnp.float32),
        ],
    )(x, A, B)


def _gather_body(row_hbm, col_hbm, p_hbm, q_hbm, pos_hbm,
                 os0, odist, orx, ory, orz,
                 rowv0, colv0, bufp0, bufq0, relb0, rxb0, ryb0, rzb0,
                 rowv1, colv1, bufp1, bufq1, relb1, rxb1, ryb1, rzb1,
                 posv,
                 sgp0, sgq0, sws0, swr0, srx0, sry0, srz0,
                 sgp1, sgq1, sws1, swr1, srx1, sry1, srz1):
    cid = lax.axis_index("c")
    sid = lax.axis_index("s")
    wid = sid * NC + cid
    base0 = wid * EW
    rowv = (rowv0, rowv1)
    colv = (colv0, colv1)
    bufp = (bufp0, bufp1)
    bufq = (bufq0, bufq1)
    relb = (relb0, relb1)
    sgp = (sgp0, sgp1)
    sgq = (sgq0, sgq1)
    sws = (sws0, sws1)
    swr = (swr0, swr1)
    relo = ((rxb0, ryb0, rzb0), (rxb1, ryb1, rzb1))
    srel = ((srx0, sry0, srz0), (srx1, sry1, srz1))
    orel3 = (orx, ory, orz)
    z16 = jnp.zeros((16,), jnp.float32)

    pltpu.sync_copy(pos_hbm, posv)

    def prefetch(j, b):
        base = base0 + j * C
        pltpu.sync_copy(row_hbm.at[pl.ds(base, C)], rowv[b])
        pltpu.sync_copy(col_hbm.at[pl.ds(base, C)], colv[b])
        pltpu.async_copy(p_hbm.at[rowv[b]], bufp[b], sgp[b])
        pltpu.async_copy(q_hbm.at[colv[b]], bufq[b], sgq[b])

    def wait_gathers(b):
        pltpu.make_async_copy(p_hbm.at[rowv[b]], bufp[b], sgp[b]).wait()
        pltpu.make_async_copy(q_hbm.at[colv[b]], bufq[b], sgq[b]).wait()

    def wait_writes(j, b):
        base = base0 + j * C
        pltpu.make_async_copy(bufp[b], os0.at[pl.ds(base, C)], sws[b]).wait()
        pltpu.make_async_copy(relb[b], odist.at[pl.ds(base, C)], swr[b]).wait()
        for kk in range(3):
            pltpu.make_async_copy(relo[b][kk], orel3[kk].at[pl.ds(base, C)],
                                  srel[b][kk]).wait()

    def compute_and_write(j, b):
        def srow(r, _):
            for d in range(D // 16):
                sl = pl.ds(d * 16, 16)
                bufp[b][r, sl] = bufp[b][r, sl] + bufq[b][r, sl]
            return 0
        lax.fori_loop(0, C, srow, 0)
        def grp(g, _):
            sl = pl.ds(g * 16, 16)
            ridx = rowv[b][sl]
            cidx = colv[b][sl]
            acc = jnp.zeros((16,), jnp.float32)
            for kk in range(3):
                kv = jnp.full((16,), kk, jnp.int32)
                a = plsc.load_gather(posv, [ridx, kv])
                c = plsc.load_gather(posv, [cidx, kv])
                r = a - c
                relo[b][kk][sl] = r
                acc = acc + r * r
            relb[b][sl] = acc
            return 0
        lax.fori_loop(0, C // 16, grp, 0)
        base = base0 + j * C
        pltpu.async_copy(bufp[b], os0.at[pl.ds(base, C)], sws[b])
        pltpu.async_copy(relb[b], odist.at[pl.ds(base, C)], swr[b])
        for kk in range(3):
            pltpu.async_copy(relo[b][kk], orel3[kk].at[pl.ds(base, C)],
                             srel[b][kk])

    prefetch(0, 0)

    def outer(k, _):
        j0 = 2 * k
        @pl.when(k > 0)
        def _():
            wait_writes(j0 - 1, 1)
        prefetch(j0 + 1, 1)
        wait_gathers(0)
        compute_and_write(j0, 0)
        wait_writes(j0, 0)
        prefetch(j0 + 2, 0)
        wait_gathers(1)
        compute_and_write(j0 + 1, 1)
        return 0

    lax.fori_loop(0, NCHUNK // 2, outer, 0)
    wait_gathers(0)
    compute_and_write(NCHUNK - 1, 0)
    wait_writes(NCHUNK - 2, 1)
    wait_writes(NCHUNK - 1, 0)


def _sc_gather(row, col, P, Q, pospad):
    f32 = jnp.float32
    i32 = jnp.int32
    slot = [
        pltpu.VMEM((C,), i32),
        pltpu.VMEM((C,), i32),
        pltpu.VMEM((C, D), f32),
        pltpu.VMEM((C, D), f32),
        pltpu.VMEM((C,), f32),
        pltpu.VMEM((C,), f32),
        pltpu.VMEM((C,), f32),
        pltpu.VMEM((C,), f32),
    ]
    sems = [pltpu.SemaphoreType.DMA] * 7
    k = functools.partial(
        pl.kernel,
        out_type=(
            jax.ShapeDtypeStruct((ES, D), f32),
            jax.ShapeDtypeStruct((ES,), f32),
            jax.ShapeDtypeStruct((ES,), f32),
            jax.ShapeDtypeStruct((ES,), f32),
            jax.ShapeDtypeStruct((ES,), f32),
        ),
        compiler_params=pltpu.CompilerParams(use_tc_tiling_on_sc=False,
                                             needs_layout_passes=False),
        mesh=plsc.VectorSubcoreMesh(core_axis_name="c", subcore_axis_name="s"),
        scratch_types=slot + slot + [pltpu.VMEM((N, 4), f32)] + sems + sems,
    )(_gather_body)
    return k(row, col, P, Q, pospad)


RB = EB // D


def _edge_body(s0_ref, dist_ref, rx_ref, ry_ref, rz_ref,
               wlast_ref, be1_ref, we2_ref, be2_ref,
               wc1_ref, bc1_ref, wc2_ref,
               em_ref, ax_ref, ay_ref, az_ref):
    x1 = s0_ref[:]
    dp = dist_ref[0]
    dpt = lax.transpose(dp, (1, 0))
    dist = jnp.concatenate(
        [dpt[:, r:r + 1] for r in range(RB)], axis=0)
    h = _silu(x1 + dist * wlast_ref[:] + be1_ref[:])
    em = _silu(_dot(h, we2_ref[:]) + be2_ref[:])
    t = _silu(_dot(em, wc1_ref[:]) + bc1_ref[:])
    cw = _dot(t, wc2_ref[:])
    cwm = jnp.concatenate(
        [cw[D * r:D * r + D, :] for r in range(RB)], axis=1)
    cwp = lax.transpose(cwm, (1, 0))
    em_ref[:] = em
    ax_ref[0] = cwp * rx_ref[0]
    ay_ref[0] = cwp * ry_ref[0]
    az_ref[0] = cwp * rz_ref[0]


def _tc_edge(s0, distp, rxp, ryp, rzp, wlast, be1, We2, be2, Wc1, bc1, wc2c):
    w0 = lambda i: (0, 0)
    pk = pl.BlockSpec((1, RB, D), lambda i: (i, 0, 0))
    return pl.pallas_call(
        _edge_body,
        grid=(ES // EB,),
        in_specs=[
            pl.BlockSpec((EB, D), lambda i: (i, 0)),
            pk, pk, pk, pk,
            pl.BlockSpec((1, D), w0),
            pl.BlockSpec((1, D), w0),
            pl.BlockSpec((D, D), w0),
            pl.BlockSpec((1, D), w0),
            pl.BlockSpec((D, D), w0),
            pl.BlockSpec((1, D), w0),
            pl.BlockSpec((D, 1), w0),
        ],
        out_specs=[
            pl.BlockSpec((EB, D), lambda i: (i, 0)),
            pk, pk, pk,
        ],
        out_shape=[
            jax.ShapeDtypeStruct((ES, D), jnp.float32),
            jax.ShapeDtypeStruct((ES // EB, RB, D), jnp.float32),
            jax.ShapeDtypeStruct((ES // EB, RB, D), jnp.float32),
            jax.ShapeDtypeStruct((ES // EB, RB, D), jnp.float32),
        ],
    )(s0, distp, rxp, ryp, rzp, wlast, be1, We2, be2, Wc1, bc1, wc2c)


def _scatter_body(*refs):
    (row_hbm, em_hbm, rx_hbm, ry_hbm, rz_hbm) = [refs[5 * t:5 * t + 5]
                                                 for t in range(SEG)][0]
    seg_refs = [refs[5 * t:5 * t + 5] for t in range(SEG)]
    (agg_out, aux_out,
     idxv0, emv0, rxv0, ryv0, rzv0,
     idxv1, emv1, rxv1, ryv1, rzv1,
     auxu, db, dba, agg_s, aux_s,
     se0, sx0, sy0, sz0,
     se1, sx1, sy1, sz1) = refs[5 * SEG:]
    cid = lax.axis_index("c")
    sid = lax.axis_index("s")
    wid = sid * NC + cid
    base0 = wid * EW
    z16 = jnp.zeros((16,), jnp.float32)
    iota16 = lax.iota(jnp.int32, 16)
    e3row = (iota16 == 3).astype(jnp.float32)

    def initaux(i, _):
        auxu[i, :] = e3row
        return 0

    lax.fori_loop(0, C, initaux, 0)

    def zrow(i, _):
        def zcol(j, _):
            db[i, pl.ds(j * 16, 16)] = z16
            return 0
        lax.fori_loop(0, D // 16, zcol, 0)
        dba[i, :] = z16
        return 0

    lax.fori_loop(0, DN, zrow, 0)

    r0 = sid * RPS

    def zdump(j, _):
        pltpu.sync_copy(db, agg_s.at[pl.ds(r0 + j * DN, DN)])
        pltpu.sync_copy(dba, aux_s.at[pl.ds(r0 + j * DN, DN)])
        return 0

    lax.fori_loop(0, RPS // DN, zdump, 0)
    plsc.subcore_barrier()

    idxv = (idxv0, idxv1)
    emv = (emv0, emv1)
    relv = ((rxv0, ryv0, rzv0), (rxv1, ryv1, rzv1))
    se = (se0, se1)
    srel = ((sx0, sy0, sz0), (sx1, sy1, sz1))

    for t in range(SEG):
        (row_hbm, em_hbm, rx_hbm, ry_hbm, rz_hbm) = seg_refs[t]
        rel_hbm = (rx_hbm, ry_hbm, rz_hbm)

        def prefetch(j, b, row_hbm=row_hbm, em_hbm=em_hbm, rel_hbm=rel_hbm):
            base = base0 + j * C
            pltpu.sync_copy(row_hbm.at[pl.ds(base, C)], idxv[b])
            pltpu.async_copy(em_hbm.at[pl.ds(base, C)], emv[b], se[b])
            for kk in range(3):
                pltpu.async_copy(rel_hbm[kk].at[pl.ds(base, C)], relv[b][kk],
                                 srel[b][kk])

        def scatter(j, b, em_hbm=em_hbm, rel_hbm=rel_hbm):
            base = base0 + j * C
            pltpu.make_async_copy(em_hbm.at[pl.ds(base, C)], emv[b],
                                  se[b]).wait()
            for kk in range(3):
                pltpu.make_async_copy(rel_hbm[kk].at[pl.ds(base, C)],
                                      relv[b][kk], srel[b][kk]).wait()
            def grp(g, _, b=b):
                sl = pl.ds(g * 16, 16)
                eidx = g * 16 + iota16
                for kk in range(3):
                    kv = jnp.full((16,), kk, jnp.int32)
                    plsc.store_scatter(auxu, [eidx, kv], relv[b][kk][sl])
                return 0
            lax.fori_loop(0, C // 16, grp, 0)
            pltpu.sync_copy(emv[b], agg_s.at[idxv[b]], add=True)
            pltpu.sync_copy(auxu, aux_s.at[idxv[b]], add=True)

        prefetch(0, 0)

        def chunk(k, _, prefetch=prefetch, scatter=scatter):
            j0 = 2 * k
            prefetch(j0 + 1, 1)
            scatter(j0, 0)
            prefetch(j0 + 2, 0)
            scatter(j0 + 1, 1)
            return 0

        lax.fori_loop(0, NCHUNK // 2, chunk, 0)
        scatter(NCHUNK - 1, 0)
    plsc.subcore_barrier()

    pltpu.sync_copy(agg_s.at[pl.ds(r0, RPS)],
                    agg_out.at[cid].at[pl.ds(r0, RPS)])
    pltpu.sync_copy(aux_s.at[pl.ds(r0, RPS)],
                    aux_out.at[cid].at[pl.ds(r0, RPS)])


def _sc_scatter(seg_args):
    f32 = jnp.float32
    k = functools.partial(
        pl.kernel,
        out_type=(
            jax.ShapeDtypeStruct((NC, N, D), f32),
            jax.ShapeDtypeStruct((NC, N, PW), f32),
        ),
        compiler_params=pltpu.CompilerParams(use_tc_tiling_on_sc=False,
                                             needs_layout_passes=False,
                                             internal_scratch_in_bytes=1 << 18),
        mesh=plsc.VectorSubcoreMesh(core_axis_name="c", subcore_axis_name="s"),
        scratch_types=(
            [pltpu.VMEM((C,), jnp.int32),
             pltpu.VMEM((C, D), f32),
             pltpu.VMEM((C,), f32),
             pltpu.VMEM((C,), f32),
             pltpu.VMEM((C,), f32)] * 2
            + [pltpu.VMEM((C, PW), f32),
               pltpu.VMEM((DN, D), f32),
               pltpu.VMEM((DN, PW), f32),
               pltpu.VMEM_SHARED((N, D), f32),
               pltpu.VMEM_SHARED((N, PW), f32)]
            + [pltpu.SemaphoreType.DMA] * 8
        ),
    )(_scatter_body)
    flat = [a for sa in seg_args for a in sa]
    return k(*flat)


def _node_body(x_ref, agg_ref, aux_ref, pos_ref,
               wn1a_ref, wn1b_ref, bn1_ref, wn2_ref, bn2_ref,
               g_ref, b_ref,
               xn_ref, pn_ref):
    xb = x_ref[:]
    agg = agg_ref[0] + agg_ref[1]
    aux = aux_ref[0] + aux_ref[1]
    cnt = aux[:, 3:4]
    inv = 1.0 / (cnt + 1e-6)
    pn_ref[:] = pos_ref[:] + aux * inv
    u = _silu(_dot(xb, wn1a_ref[:]) + _dot(agg, wn1b_ref[:]) + bn1_ref[:])
    v = xb + _dot(u, wn2_ref[:]) + bn2_ref[:]
    mu = jnp.mean(v, axis=1, keepdims=True)
    vc = v - mu
    var = jnp.mean(vc * vc, axis=1, keepdims=True)
    xn_ref[:] = vc * lax.rsqrt(var + 1e-5) * g_ref[:] + b_ref[:]


def _tc_node(x, aggP, auxP, pospad,
             Wn1a, Wn1b, bn1, Wn2, bn2, gamma, beta):
    w0 = lambda i: (0, 0)
    return pl.pallas_call(
        _node_body,
        grid=(N // NB,),
        in_specs=[
            pl.BlockSpec((NB, D), lambda i: (i, 0)),
            pl.BlockSpec((NC, NB, D), lambda i: (0, i, 0)),
            pl.BlockSpec((NC, NB, PW), lambda i: (0, i, 0)),
            pl.BlockSpec((NB, PW), lambda i: (i, 0)),
            pl.BlockSpec((D, D), w0),
            pl.BlockSpec((D, D), w0),
            pl.BlockSpec((1, D), w0),
            pl.BlockSpec((D, D), w0),
            pl.BlockSpec((1, D), w0),
            pl.BlockSpec((1, D), w0),
            pl.BlockSpec((1, D), w0),
        ],
        out_specs=[
            pl.BlockSpec((NB, D), lambda i: (i, 0)),
            pl.BlockSpec((NB, PW), lambda i: (i, 0)),
        ],
        out_shape=[
            jax.ShapeDtypeStruct((N, D), jnp.float32),
            jax.ShapeDtypeStruct((N, PW), jnp.float32),
        ],
    )(x, aggP, auxP, pospad,
      Wn1a, Wn1b, bn1, Wn2, bn2, gamma, beta)


def kernel(x, pos, edge_index, We1, be1, We2, be2, Wn1, bn1, Wn2, bn2,
           Wc1, bc1, Wc2, gamma, beta):
    row = edge_index[0]
    col = edge_index[1]
    pospad4 = jnp.pad(pos, ((0, 0), (0, 1)))
    pospad = jnp.pad(pos, ((0, 0), (0, PW - 3)))
    A = We1[:D]
    B = We1[D:2 * D]
    wlast = We1[2 * D:2 * D + 1]
    P, Q = _tc_precompute(x, A, B)
    pshape = (ES // EB, RB, D)
    seg_args = []
    for sgi in range(SEG):
        row_s = lax.slice(row, (sgi * ES,), ((sgi + 1) * ES,))
        col_s = lax.slice(col, (sgi * ES,), ((sgi + 1) * ES,))
        s0, dist, rx, ry, rz = _sc_gather(row_s, col_s, P, Q, pospad4)
        em, axp, ayp, azp = _tc_edge(s0, dist.reshape(pshape),
                                     rx.reshape(pshape),
                                     ry.reshape(pshape),
                                     rz.reshape(pshape),
                                     wlast, be1.reshape(1, D), We2,
                                     be2.reshape(1, D), Wc1,
                                     bc1.reshape(1, D), Wc2)
        seg_args.append((row_s, em, axp.reshape(ES), ayp.reshape(ES),
                         azp.reshape(ES)))
    aggP, auxP = _sc_scatter(seg_args)
    xn, pn = _tc_node(x, aggP, auxP, pospad,
                      Wn1[:D], Wn1[D:], bn1.reshape(1, D), Wn2,
                      bn2.reshape(1, D), gamma.reshape(1, D),
                      beta.reshape(1, D))
    return xn, pn[:, :3]
